# initial kernel scaffold (unmeasured)
import jax
import jax.numpy as jnp
from jax import lax
from jax.experimental import pallas as pl
from jax.experimental.pallas import tpu as pltpu


def kernel(
    x,
):
    def body(*refs):
        pass

    out_shape = jax.ShapeDtypeStruct(..., jnp.float32)
    return pl.pallas_call(body, out_shape=out_shape)(...)


# baseline (device time: 96780 ns/iter reference)
import jax
import jax.numpy as jnp
from jax import lax
from jax.experimental import pallas as pl
from jax.experimental.pallas import tpu as pltpu

N_DEV = 8
N_HOPS = N_DEV // 2


def kernel(x):
    m_per, n = x.shape

    def gray(t):
        t = t % N_DEV
        return jnp.where(t < 4, t, 11 - t)

    def body(x_ref, out_ref, fwd_send, fwd_recv, bwd_send, bwd_recv):
        my_pos = lax.axis_index("i")
        r = gray(my_pos)
        nxt = gray(r + 1)
        prv = gray(r - 1)

        barrier_sem = pltpu.get_barrier_semaphore()
        for nbr in (nxt, prv):
            pl.semaphore_signal(
                barrier_sem, inc=1,
                device_id=(nbr,), device_id_type=pl.DeviceIdType.MESH,
            )
        pl.semaphore_wait(barrier_sem, 2)

        out_ref[pl.ds(my_pos * m_per, m_per), :] = x_ref[:, :]

        half = m_per // 2
        for h in range(N_HOPS):
            of = gray(r - h)
            ob = gray(r + h)
            if h < N_HOPS - 1:
                f_start, f_rows = of * m_per, m_per
                b_start, b_rows = ob * m_per, m_per
            else:
                f_start, f_rows = of * m_per, half
                b_start, b_rows = ob * m_per + half, half
            fwd = pltpu.make_async_remote_copy(
                src_ref=out_ref.at[pl.ds(f_start, f_rows), :],
                dst_ref=out_ref.at[pl.ds(f_start, f_rows), :],
                send_sem=fwd_send.at[h],
                recv_sem=fwd_recv.at[h],
                device_id=(nxt,),
                device_id_type=pl.DeviceIdType.MESH,
            )
            bwd = pltpu.make_async_remote_copy(
                src_ref=out_ref.at[pl.ds(b_start, b_rows), :],
                dst_ref=out_ref.at[pl.ds(b_start, b_rows), :],
                send_sem=bwd_send.at[h],
                recv_sem=bwd_recv.at[h],
                device_id=(prv,),
                device_id_type=pl.DeviceIdType.MESH,
            )
            fwd.start()
            bwd.start()
            fwd.wait()
            bwd.wait()

    return pl.pallas_call(
        body,
        out_shape=jax.ShapeDtypeStruct((N_DEV * m_per, n), x.dtype),
        in_specs=[pl.BlockSpec(memory_space=pltpu.VMEM)],
        out_specs=pl.BlockSpec(memory_space=pltpu.VMEM),
        scratch_shapes=[
            pltpu.SemaphoreType.DMA((N_HOPS,)),
            pltpu.SemaphoreType.DMA((N_HOPS,)),
            pltpu.SemaphoreType.DMA((N_HOPS,)),
            pltpu.SemaphoreType.DMA((N_HOPS,)),
        ],
        compiler_params=pltpu.CompilerParams(collective_id=0),
    )(x)


# device time: 91332 ns/iter; 1.0597x vs baseline; 1.0597x over previous
import jax
import jax.numpy as jnp
from jax import lax
from jax.experimental import pallas as pl
from jax.experimental.pallas import tpu as pltpu

N_DEV = 8
N_HOPS = N_DEV // 2
N_SEG = 2


def kernel(x):
    m_per, n = x.shape
    seg = m_per // N_SEG

    def gray(t):
        t = t % N_DEV
        return jnp.where(t < 4, t, 11 - t)

    def segs(h):
        return (0, 1) if h < N_HOPS - 1 else (0,)

    def body(x_ref, out_ref, fwd_send, fwd_recv, bwd_send, bwd_recv):
        my_pos = lax.axis_index("i")
        r = gray(my_pos)
        nxt = gray(r + 1)
        prv = gray(r - 1)

        barrier_sem = pltpu.get_barrier_semaphore()
        for nbr in (nxt, prv):
            pl.semaphore_signal(
                barrier_sem, inc=1,
                device_id=(nbr,), device_id_type=pl.DeviceIdType.MESH,
            )
        pl.semaphore_wait(barrier_sem, 2)

        def desc(src, row0, rows, sems_s, sems_r, h, k, target):
            return pltpu.make_async_remote_copy(
                src_ref=src,
                dst_ref=out_ref.at[pl.ds(row0, rows), :],
                send_sem=sems_s.at[h, k],
                recv_sem=sems_r.at[h, k],
                device_id=(target,),
                device_id_type=pl.DeviceIdType.MESH,
            )

        df, db = {}, {}
        for h in range(N_HOPS):
            for k in segs(h):
                if h == 0:
                    of_row = my_pos * m_per + k * seg
                    ob_row = of_row
                    f_src = x_ref.at[pl.ds(k * seg, seg), :]
                    b_src = x_ref.at[pl.ds(k * seg, seg), :]
                elif h < N_HOPS - 1:
                    of_row = gray(r - h) * m_per + k * seg
                    ob_row = gray(r + h) * m_per + k * seg
                    f_src = out_ref.at[pl.ds(of_row, seg), :]
                    b_src = out_ref.at[pl.ds(ob_row, seg), :]
                else:
                    of_row = gray(r - h) * m_per
                    ob_row = gray(r + h) * m_per + seg
                    f_src = out_ref.at[pl.ds(of_row, seg), :]
                    b_src = out_ref.at[pl.ds(ob_row, seg), :]
                df[h, k] = desc(f_src, of_row, seg, fwd_send, fwd_recv, h, k, nxt)
                db[h, k] = desc(b_src, ob_row, seg, bwd_send, bwd_recv, h, k, prv)

        for k in (0, 1):
            df[0, k].start()
            db[0, k].start()
        out_ref[pl.ds(my_pos * m_per, m_per), :] = x_ref[:, :]

        for h in range(1, N_HOPS):
            for k in segs(h):
                df[h - 1, k].wait_recv()
                df[h, k].start()
                db[h - 1, k].wait_recv()
                db[h, k].start()

        df[2, 1].wait_recv()
        db[2, 1].wait_recv()
        df[3, 0].wait_recv()
        db[3, 0].wait_recv()

        for d in (*df.values(), *db.values()):
            d.wait_send()

    return pl.pallas_call(
        body,
        out_shape=jax.ShapeDtypeStruct((N_DEV * m_per, n), x.dtype),
        in_specs=[pl.BlockSpec(memory_space=pltpu.VMEM)],
        out_specs=pl.BlockSpec(memory_space=pltpu.VMEM),
        scratch_shapes=[
            pltpu.SemaphoreType.DMA((N_HOPS, N_SEG)),
            pltpu.SemaphoreType.DMA((N_HOPS, N_SEG)),
            pltpu.SemaphoreType.DMA((N_HOPS, N_SEG)),
            pltpu.SemaphoreType.DMA((N_HOPS, N_SEG)),
        ],
        compiler_params=pltpu.CompilerParams(collective_id=0),
    )(x)
